# baseline (speedup 1.0000x reference)
import functools

import jax
import jax.numpy as jnp
from jax import lax
from jax.experimental import pallas as pl
from jax.experimental.pallas import tpu as pltpu
from jax.experimental.pallas import tpu_sc as plsc

_DIM = 16
_N = 100000
_B = 1024
_K = 5
_T = 0.1

_R = 2048
_NPAD = 100352
_GRID = _NPAD // _R
_NEG = -1e30
_IBIG = 2**30
_BCORES = 2
_BB = _B // _BCORES


def _tc_body(q_ref, mT_ref, w_ref, i_ref, d_ref, rv_ref, ri_ref):
    step = pl.program_id(1)

    @pl.when(step == 0)
    def _init():
        rv_ref[...] = jnp.full((_BB, 128), _NEG, jnp.float32)
        ri_ref[...] = jnp.full((_BB, 128), _IBIG, jnp.int32)

    q = q_ref[...]
    qinv = 1.0 / jnp.maximum(jnp.sqrt(jnp.sum(q * q, axis=1, keepdims=True)), 1e-12)
    m = mT_ref[...]
    minv = 1.0 / jnp.maximum(jnp.sqrt(jnp.sum(m * m, axis=0, keepdims=True)), 1e-12)
    qb = (q * qinv).astype(jnp.bfloat16)
    mb = (m * minv).astype(jnp.bfloat16)
    sim = lax.dot_general(qb, mb, (((1,), (0,)), ((), ())),
                          preferred_element_type=jnp.float32)
    gcol = step * _R + lax.broadcasted_iota(jnp.int32, (_BB, _R), 1)
    sim = jnp.where(gcol < _N, sim, _NEG)

    cv = jnp.concatenate([sim, rv_ref[...]], axis=1)
    ci = jnp.concatenate([gcol, ri_ref[...]], axis=1)
    lane = lax.broadcasted_iota(jnp.int32, (_BB, 128), 1)
    nv = jnp.full((_BB, 128), _NEG, jnp.float32)
    ni = jnp.full((_BB, 128), _IBIG, jnp.int32)
    for j in range(_K):
        mj = jnp.max(cv, axis=1, keepdims=True)
        cand = jnp.where(cv == mj, ci, _IBIG)
        sel = jnp.min(cand, axis=1, keepdims=True)
        nv = jnp.where(lane == j, mj, nv)
        ni = jnp.where(lane == j, sel, ni)
        cv = jnp.where(ci == sel, _NEG, cv)
    rv_ref[...] = nv
    ri_ref[...] = ni

    @pl.when(step == _GRID - 1)
    def _fin():
        vmax = jnp.max(nv, axis=1, keepdims=True)
        e = jnp.where(lane < _K, jnp.exp((nv - vmax) / _T), 0.0)
        s = jnp.sum(e, axis=1, keepdims=True)
        w_ref[...] = e / s
        i_ref[...] = ni
        d_ref[...] = jnp.broadcast_to(1.0 - vmax, (_BB, 128))


def _tc_topk(q, mT):
    return pl.pallas_call(
        _tc_body,
        grid=(_BCORES, _GRID),
        in_specs=[
            pl.BlockSpec((_BB, _DIM), lambda b, i: (b, 0)),
            pl.BlockSpec((_DIM, _R), lambda b, i: (0, i)),
        ],
        out_specs=[pl.BlockSpec((_BB, 128), lambda b, i: (b, 0))] * 3,
        out_shape=[
            jax.ShapeDtypeStruct((_B, 128), jnp.float32),
            jax.ShapeDtypeStruct((_B, 128), jnp.int32),
            jax.ShapeDtypeStruct((_B, 128), jnp.float32),
        ],
        scratch_shapes=[
            pltpu.VMEM((_BB, 128), jnp.float32),
            pltpu.VMEM((_BB, 128), jnp.int32),
        ],
        compiler_params=pltpu.CompilerParams(
            dimension_semantics=("parallel", "arbitrary")),
    )(q, mT)


_NW = 32
_QPW = _B // _NW
_RPW = _QPW * _K
_CH = 2
_QPC = _QPW // _CH
_RPC = _RPW // _CH


def _lanesum(x):
    lanes = lax.iota(jnp.int32, _DIM)
    for sh in (8, 4, 2, 1):
        x = x + x.at[lanes ^ sh].get(mode="promise_in_bounds")
    return x


def _rsqrt_v(ssv):
    bits = lax.bitcast_convert_type(ssv, jnp.int32)
    y = lax.bitcast_convert_type(jnp.int32(0x5F3759DF) - (bits >> 1), jnp.float32)
    for _ in range(3):
        y = y * (1.5 - 0.5 * ssv * y * y)
    return jnp.minimum(y, 1e12)


def _sc_body(mem_hbm, idx_hbm, w_hbm, out_hbm, idx_v, rows_v, w_v, out_v, sem):
    wid = lax.axis_index("s") * 2 + lax.axis_index("c")
    for c in range(_CH):
        rbase = wid * _RPW + c * _RPC
        qbase = wid * _QPW + c * _QPC
        pltpu.sync_copy(idx_hbm.at[pl.ds(rbase, _RPC)], idx_v)
        pltpu.sync_copy(w_hbm.at[pl.ds(rbase, _RPC), :], w_v)
        pltpu.async_copy(mem_hbm.at[idx_v], rows_v, sem).wait()

        def qstep(i, carry):
            acc = jnp.zeros((_DIM,), jnp.float32)
            for j in range(_K):
                r = i * _K + j
                row = rows_v[r]
                wv = w_v[r]
                inv = _rsqrt_v(_lanesum(row * row))
                acc = acc + wv * inv * row
            inv2 = _rsqrt_v(_lanesum(acc * acc))
            out_v[i] = acc * inv2
            return carry

        lax.fori_loop(0, _QPC, qstep, 0)
        pltpu.sync_copy(out_v, out_hbm.at[pl.ds(qbase, _QPC), :])


@functools.cache
def _sc_retrieve():
    return pl.kernel(
        _sc_body,
        mesh=plsc.VectorSubcoreMesh(core_axis_name="c", subcore_axis_name="s"),
        out_type=jax.ShapeDtypeStruct((_B, _DIM), jnp.float32),
        scratch_types=[
            pltpu.VMEM((_RPC,), jnp.int32),
            pltpu.VMEM((_RPC, _DIM), jnp.float32),
            pltpu.VMEM((_RPC, _DIM), jnp.float32),
            pltpu.VMEM((_QPC, _DIM), jnp.float32),
            pltpu.SemaphoreType.DMA,
        ],
        compiler_params=pltpu.CompilerParams(use_tc_tiling_on_sc=False),
    )


def kernel(query, memory):
    memT = jnp.zeros((_DIM, _NPAD), jnp.float32).at[:, :_N].set(memory.T)
    w128, i128, d128 = _tc_topk(query, memT)
    weights = w128[:, :_K]
    idxflat = i128[:, :_K].reshape(-1)
    wexp = jnp.broadcast_to(weights.reshape(_B * _K, 1), (_B * _K, _DIM))
    retrieved = _sc_retrieve()(memory, idxflat, wexp)
    return retrieved, d128[:, 0], weights


# speedup vs baseline: 1.5236x; 1.5236x over previous
import functools

import jax
import jax.numpy as jnp
from jax import lax
from jax.experimental import pallas as pl
from jax.experimental.pallas import tpu as pltpu
from jax.experimental.pallas import tpu_sc as plsc

_DIM = 16
_N = 100000
_B = 1024
_K = 5
_T = 0.1

_R = 2048
_NPAD = 100352
_GRID = _NPAD // _R
_CHUNKS = _R // 128
_NEG = -1e30
_IBIGF = float(2**24)


def _tc_body(q_ref, mT_ref, w_ref, i_ref, d_ref, *mg_refs):
    mv_refs, gi_refs = mg_refs[:_K], mg_refs[_K:]
    step = pl.program_id(0)

    @pl.when(step == 0)
    def _init():
        for k in range(_K):
            mv_refs[k][...] = jnp.full((_B, 128), _NEG, jnp.float32)
            gi_refs[k][...] = jnp.full((_B, 128), _IBIGF, jnp.float32)

    q = q_ref[...]
    qinv = 1.0 / jnp.maximum(jnp.sqrt(jnp.sum(q * q, axis=1, keepdims=True)), 1e-12)
    m = mT_ref[...]
    minv = 1.0 / jnp.maximum(jnp.sqrt(jnp.sum(m * m, axis=0, keepdims=True)), 1e-12)
    qb = (q * qinv).astype(jnp.bfloat16)
    mb = (m * minv).astype(jnp.bfloat16)
    sim = lax.dot_general(qb, mb, (((1,), (0,)), ((), ())),
                          preferred_element_type=jnp.float32)

    lane_f = lax.broadcasted_iota(jnp.int32, (_B, 128), 1).astype(jnp.float32)
    base = (step * _R).astype(jnp.float32)
    M = [mv_refs[k][...] for k in range(_K)]
    G = [gi_refs[k][...] for k in range(_K)]
    for c in range(_CHUNKS):
        v = sim[:, c * 128:(c + 1) * 128]
        gv = lane_f + (base + float(c * 128))
        v = jnp.where(gv < float(_N), v, _NEG)
        for k in range(_K):
            b = v > M[k]
            nv_ = jnp.maximum(v, M[k])
            ng_ = jnp.where(b, gv, G[k])
            lv_ = jnp.minimum(v, M[k])
            lg_ = jnp.where(b, G[k], gv)
            M[k], G[k] = nv_, ng_
            v, gv = lv_, lg_
    for k in range(_K):
        mv_refs[k][...] = M[k]
        gi_refs[k][...] = G[k]

    @pl.when(step == _GRID - 1)
    def _fin():
        cv = jnp.concatenate(M, axis=1)
        ci = jnp.concatenate(G, axis=1)
        lane = lax.broadcasted_iota(jnp.int32, (_B, 128), 1)
        nv = jnp.full((_B, 128), _NEG, jnp.float32)
        ni = jnp.full((_B, 128), _IBIGF, jnp.float32)
        for j in range(_K):
            mj = jnp.max(cv, axis=1, keepdims=True)
            cand = jnp.where(cv == mj, ci, _IBIGF)
            sel = jnp.min(cand, axis=1, keepdims=True)
            nv = jnp.where(lane == j, mj, nv)
            ni = jnp.where(lane == j, sel, ni)
            if j < _K - 1:
                cv = jnp.where(ci == sel, _NEG, cv)
        vmax = jnp.max(nv, axis=1, keepdims=True)
        e = jnp.where(lane < _K, jnp.exp((nv - vmax) / _T), 0.0)
        s = jnp.sum(e, axis=1, keepdims=True)
        w_ref[...] = e / s
        i_ref[...] = ni.astype(jnp.int32)
        d_ref[...] = jnp.broadcast_to(1.0 - vmax, (_B, 128))


def _tc_topk(q, mT):
    return pl.pallas_call(
        _tc_body,
        grid=(_GRID,),
        in_specs=[
            pl.BlockSpec((_B, _DIM), lambda i: (0, 0)),
            pl.BlockSpec((_DIM, _R), lambda i: (0, i)),
        ],
        out_specs=[pl.BlockSpec((_B, 128), lambda i: (0, 0))] * 3,
        out_shape=[
            jax.ShapeDtypeStruct((_B, 128), jnp.float32),
            jax.ShapeDtypeStruct((_B, 128), jnp.int32),
            jax.ShapeDtypeStruct((_B, 128), jnp.float32),
        ],
        scratch_shapes=[pltpu.VMEM((_B, 128), jnp.float32)] * (2 * _K),
    )(q, mT)


_NW = 32
_QPW = _B // _NW
_RPW = _QPW * _K
_CH = 2
_QPC = _QPW // _CH
_RPC = _RPW // _CH


def _lanesum(x):
    lanes = lax.iota(jnp.int32, _DIM)
    for sh in (8, 4, 2, 1):
        x = x + x.at[lanes ^ sh].get(mode="promise_in_bounds")
    return x


def _rsqrt_v(ssv):
    bits = lax.bitcast_convert_type(ssv, jnp.int32)
    y = lax.bitcast_convert_type(jnp.int32(0x5F3759DF) - (bits >> 1), jnp.float32)
    for _ in range(3):
        y = y * (1.5 - 0.5 * ssv * y * y)
    return jnp.minimum(y, 1e12)


def _sc_body(mem_hbm, idx_hbm, w_hbm, out_hbm, idx_v, rows_v, w_v, out_v, sem):
    wid = lax.axis_index("s") * 2 + lax.axis_index("c")
    for c in range(_CH):
        rbase = wid * _RPW + c * _RPC
        qbase = wid * _QPW + c * _QPC
        pltpu.sync_copy(idx_hbm.at[pl.ds(rbase, _RPC)], idx_v)
        pltpu.sync_copy(w_hbm.at[pl.ds(rbase, _RPC), :], w_v)
        pltpu.async_copy(mem_hbm.at[idx_v], rows_v, sem).wait()

        def qstep(i, carry):
            acc = jnp.zeros((_DIM,), jnp.float32)
            for j in range(_K):
                r = i * _K + j
                row = rows_v[r]
                wv = w_v[r]
                inv = _rsqrt_v(_lanesum(row * row))
                acc = acc + wv * inv * row
            inv2 = _rsqrt_v(_lanesum(acc * acc))
            out_v[i] = acc * inv2
            return carry

        lax.fori_loop(0, _QPC, qstep, 0)
        pltpu.sync_copy(out_v, out_hbm.at[pl.ds(qbase, _QPC), :])


@functools.cache
def _sc_retrieve():
    return pl.kernel(
        _sc_body,
        mesh=plsc.VectorSubcoreMesh(core_axis_name="c", subcore_axis_name="s"),
        out_type=jax.ShapeDtypeStruct((_B, _DIM), jnp.float32),
        scratch_types=[
            pltpu.VMEM((_RPC,), jnp.int32),
            pltpu.VMEM((_RPC, _DIM), jnp.float32),
            pltpu.VMEM((_RPC, _DIM), jnp.float32),
            pltpu.VMEM((_QPC, _DIM), jnp.float32),
            pltpu.SemaphoreType.DMA,
        ],
        compiler_params=pltpu.CompilerParams(use_tc_tiling_on_sc=False),
    )


def kernel(query, memory):
    memT = jnp.zeros((_DIM, _NPAD), jnp.float32).at[:, :_N].set(memory.T)
    w128, i128, d128 = _tc_topk(query, memT)
    weights = w128[:, :_K]
    idxflat = i128[:, :_K].reshape(-1)
    wexp = jnp.broadcast_to(weights.reshape(_B * _K, 1), (_B * _K, _DIM))
    retrieved = _sc_retrieve()(memory, idxflat, wexp)
    return retrieved, d128[:, 0], weights


# speedup vs baseline: 1.5779x; 1.0357x over previous
import functools

import jax
import jax.numpy as jnp
from jax import lax
from jax.experimental import pallas as pl
from jax.experimental.pallas import tpu as pltpu
from jax.experimental.pallas import tpu_sc as plsc

_DIM = 16
_N = 100000
_B = 1024
_K = 5
_T = 0.1

_R = 2048
_NPAD = 100352
_GRID = _NPAD // _R
_CHUNKS = _R // 128
_NEG = -1e30
_IBIGF = float(2**24)


def _tc_body(q_ref, mT_ref, w_ref, i_ref, d_ref, *mg_refs):
    mv_refs, gi_refs = mg_refs[:_K], mg_refs[_K:]
    step = pl.program_id(0)

    @pl.when(step == 0)
    def _init():
        for k in range(_K):
            mv_refs[k][...] = jnp.full((_B, 128), _NEG, jnp.float32)
            gi_refs[k][...] = jnp.full((_B, 128), _IBIGF, jnp.float32)

    q = q_ref[...]
    qinv = 1.0 / jnp.maximum(jnp.sqrt(jnp.sum(q * q, axis=1, keepdims=True)), 1e-12)
    m = mT_ref[...]
    minv = 1.0 / jnp.maximum(jnp.sqrt(jnp.sum(m * m, axis=0, keepdims=True)), 1e-12)
    qb = (q * qinv).astype(jnp.bfloat16)
    mb = (m * minv).astype(jnp.bfloat16)
    sim = lax.dot_general(qb, mb, (((1,), (0,)), ((), ())),
                          preferred_element_type=jnp.float32)

    basef = step.astype(jnp.float32)
    _TILE = 64
    for t in range(_B // _TILE):
        rs = slice(t * _TILE, (t + 1) * _TILE)
        M = [mv_refs[k][rs, :] for k in range(_K)]
        G = [gi_refs[k][rs, :] for k in range(_K)]
        lane_f = lax.broadcasted_iota(jnp.int32, (_TILE, 128), 1).astype(jnp.float32)
        for c in range(_CHUNKS):
            v = sim[t * _TILE:(t + 1) * _TILE, c * 128:(c + 1) * 128]
            gv = basef * float(_CHUNKS) + float(c)
            if c >= 13:
                thresh = float(_N) - basef * float(_R) - float(c * 128)
                v = jnp.where(lane_f >= thresh, _NEG, v)
            for k in range(_K):
                b = v > M[k]
                nv_ = jnp.maximum(v, M[k])
                ng_ = jnp.where(b, gv, G[k])
                lv_ = jnp.minimum(v, M[k])
                lg_ = jnp.where(b, G[k], gv)
                M[k], G[k] = nv_, ng_
                v, gv = lv_, lg_
        for k in range(_K):
            mv_refs[k][rs, :] = M[k]
            gi_refs[k][rs, :] = G[k]

    @pl.when(step == _GRID - 1)
    def _fin():
        lane_r = lax.broadcasted_iota(jnp.int32, (_B, 128), 1).astype(jnp.float32)
        cv = jnp.concatenate([mv_refs[k][...] for k in range(_K)], axis=1)
        ci = jnp.concatenate(
            [gi_refs[k][...] * 128.0 + lane_r for k in range(_K)], axis=1)
        lane = lax.broadcasted_iota(jnp.int32, (_B, 128), 1)
        nv = jnp.full((_B, 128), _NEG, jnp.float32)
        ni = jnp.full((_B, 128), _IBIGF, jnp.float32)
        for j in range(_K):
            mj = jnp.max(cv, axis=1, keepdims=True)
            cand = jnp.where(cv == mj, ci, _IBIGF)
            sel = jnp.min(cand, axis=1, keepdims=True)
            nv = jnp.where(lane == j, mj, nv)
            ni = jnp.where(lane == j, sel, ni)
            if j < _K - 1:
                cv = jnp.where(ci == sel, _NEG, cv)
        vmax = jnp.max(nv, axis=1, keepdims=True)
        e = jnp.where(lane < _K, jnp.exp((nv - vmax) / _T), 0.0)
        s = jnp.sum(e, axis=1, keepdims=True)
        w_ref[...] = e / s
        i_ref[...] = ni.astype(jnp.int32)
        d_ref[...] = jnp.broadcast_to(1.0 - vmax, (_B, 128))


def _tc_topk(q, mT):
    return pl.pallas_call(
        _tc_body,
        grid=(_GRID,),
        in_specs=[
            pl.BlockSpec((_B, _DIM), lambda i: (0, 0)),
            pl.BlockSpec((_DIM, _R), lambda i: (0, i)),
        ],
        out_specs=[pl.BlockSpec((_B, 128), lambda i: (0, 0))] * 3,
        out_shape=[
            jax.ShapeDtypeStruct((_B, 128), jnp.float32),
            jax.ShapeDtypeStruct((_B, 128), jnp.int32),
            jax.ShapeDtypeStruct((_B, 128), jnp.float32),
        ],
        scratch_shapes=[pltpu.VMEM((_B, 128), jnp.float32)] * (2 * _K),
    )(q, mT)


_NW = 32
_QPW = _B // _NW
_RPW = _QPW * _K
_CH = 2
_QPC = _QPW // _CH
_RPC = _RPW // _CH


def _lanesum(x):
    lanes = lax.iota(jnp.int32, _DIM)
    for sh in (8, 4, 2, 1):
        x = x + x.at[lanes ^ sh].get(mode="promise_in_bounds")
    return x


def _rsqrt_v(ssv):
    bits = lax.bitcast_convert_type(ssv, jnp.int32)
    y = lax.bitcast_convert_type(jnp.int32(0x5F3759DF) - (bits >> 1), jnp.float32)
    for _ in range(3):
        y = y * (1.5 - 0.5 * ssv * y * y)
    return jnp.minimum(y, 1e12)


def _sc_body(mem_hbm, idx_hbm, w_hbm, out_hbm, idx_v, rows_v, w_v, out_v, sem):
    wid = lax.axis_index("s") * 2 + lax.axis_index("c")
    for c in range(_CH):
        rbase = wid * _RPW + c * _RPC
        qbase = wid * _QPW + c * _QPC
        pltpu.sync_copy(idx_hbm.at[pl.ds(rbase, _RPC)], idx_v)
        pltpu.sync_copy(w_hbm.at[pl.ds(rbase, _RPC), :], w_v)
        pltpu.async_copy(mem_hbm.at[idx_v], rows_v, sem).wait()

        def qstep(i, carry):
            acc = jnp.zeros((_DIM,), jnp.float32)
            for j in range(_K):
                r = i * _K + j
                row = rows_v[r]
                wv = w_v[r]
                inv = _rsqrt_v(_lanesum(row * row))
                acc = acc + wv * inv * row
            inv2 = _rsqrt_v(_lanesum(acc * acc))
            out_v[i] = acc * inv2
            return carry

        lax.fori_loop(0, _QPC, qstep, 0)
        pltpu.sync_copy(out_v, out_hbm.at[pl.ds(qbase, _QPC), :])


@functools.cache
def _sc_retrieve():
    return pl.kernel(
        _sc_body,
        mesh=plsc.VectorSubcoreMesh(core_axis_name="c", subcore_axis_name="s"),
        out_type=jax.ShapeDtypeStruct((_B, _DIM), jnp.float32),
        scratch_types=[
            pltpu.VMEM((_RPC,), jnp.int32),
            pltpu.VMEM((_RPC, _DIM), jnp.float32),
            pltpu.VMEM((_RPC, _DIM), jnp.float32),
            pltpu.VMEM((_QPC, _DIM), jnp.float32),
            pltpu.SemaphoreType.DMA,
        ],
        compiler_params=pltpu.CompilerParams(use_tc_tiling_on_sc=False),
    )


def kernel(query, memory):
    memT = jnp.zeros((_DIM, _NPAD), jnp.float32).at[:, :_N].set(memory.T)
    w128, i128, d128 = _tc_topk(query, memT)
    weights = w128[:, :_K]
    idxflat = i128[:, :_K].reshape(-1)
    wexp = jnp.broadcast_to(weights.reshape(_B * _K, 1), (_B * _K, _DIM))
    retrieved = _sc_retrieve()(memory, idxflat, wexp)
    return retrieved, d128[:, 0], weights
